# baseline (speedup 1.0000x reference)
import jax
import jax.numpy as jnp
from jax import lax
from jax.experimental import pallas as pl

B = 4096
M = 16
D = 64
N = M * M
BLK = 512


def _fused_body(x_ref, wt_ref, o_ref):
    wt = wt_ref[...]
    w2 = jnp.sum(wt * wt, axis=0, keepdims=True)
    s = w2 - 2.0 * jnp.dot(
        x_ref[...], wt, preferred_element_type=jnp.float32,
        precision=lax.Precision.HIGHEST)
    cols = []
    for g in range(M):
        grp = s[:, g * M:(g + 1) * M]
        mn = jnp.min(grp, axis=1, keepdims=True)
        io = lax.broadcasted_iota(jnp.int32, (BLK, M), 1)
        idx = jnp.min(jnp.where(grp == mn, io, M), axis=1, keepdims=True)
        cols.append(idx)
    o_ref[...] = jnp.concatenate(cols, axis=1)


def kernel(x, weights):
    wt = weights.reshape(N, D).T
    return pl.pallas_call(
        _fused_body,
        grid=(B // BLK,),
        in_specs=[
            pl.BlockSpec((BLK, D), lambda i: (i, 0)),
            pl.BlockSpec((D, N), lambda i: (0, 0)),
        ],
        out_specs=pl.BlockSpec((BLK, M), lambda i: (i, 0)),
        out_shape=jax.ShapeDtypeStruct((B, M), jnp.int32),
    )(x, wt)


# speedup vs baseline: 1.8984x; 1.8984x over previous
import functools

import jax
import jax.numpy as jnp
from jax import lax
from jax.experimental import pallas as pl
from jax.experimental.pallas import tpu as pltpu
from jax.experimental.pallas import tpu_sc as plsc

B = 4096
M = 16
D = 64
N = M * M
NC = 2
NS = 16
NW = NC * NS
ROWS = B // NW
BLK = 1024


def _nt_dot(a, b):
    return lax.dot_general(
        a, b, (((1,), (1,)), ((), ())),
        preferred_element_type=jnp.float32,
        precision=lax.Precision.HIGHEST)


def _scores_body(x_ref, w_ref, s_ref):
    w3 = w_ref[...]
    wt = jnp.concatenate([w3[:, k, :] for k in range(M)], axis=0)
    ones = jnp.ones((8, D), jnp.float32)
    w2 = _nt_dot(ones, wt * wt)[0:1, :]
    s_ref[...] = w2 - 2.0 * _nt_dot(x_ref[...], wt)


def _tc_scores(x, weights):
    return pl.pallas_call(
        _scores_body,
        grid=(B // BLK,),
        in_specs=[
            pl.BlockSpec((BLK, D), lambda i: (i, 0)),
            pl.BlockSpec((M, M, D), lambda i: (0, 0, 0)),
        ],
        out_specs=pl.BlockSpec((BLK, N), lambda i: (i, 0)),
        out_shape=jax.ShapeDtypeStruct((B, N), jnp.float32),
    )(x, weights)


def _argmin_body(s_hbm, o_hbm, s_v, o_v):
    wid = lax.axis_index("s") * NC + lax.axis_index("c")
    base = wid * ROWS
    pltpu.sync_copy(s_hbm.at[pl.ds(base, ROWS)], s_v)

    def row(r, carry):
        best = s_v[r, pl.ds(0, M)]
        bidx = jnp.zeros((M,), jnp.int32)
        for k in range(1, M):
            v = s_v[r, pl.ds(k * M, M)]
            m = v < best
            best = jnp.where(m, v, best)
            bidx = jnp.where(m, jnp.int32(k), bidx)
        o_v[r, pl.ds(0, M)] = bidx
        return carry

    lax.fori_loop(0, ROWS, row, 0)
    pltpu.sync_copy(o_v, o_hbm.at[pl.ds(base, ROWS)])


@functools.cache
def _sc_argmin():
    return pl.kernel(
        _argmin_body,
        out_type=jax.ShapeDtypeStruct((B, M), jnp.int32),
        mesh=plsc.VectorSubcoreMesh(core_axis_name="c", subcore_axis_name="s"),
        scratch_types=[
            pltpu.VMEM((ROWS, N), jnp.float32),
            pltpu.VMEM((ROWS, M), jnp.int32),
        ],
    )


def kernel(x, weights):
    scores = _tc_scores(x, weights)
    return _sc_argmin()(scores)
